# baseline (speedup 1.0000x reference)
import functools

import jax
import jax.numpy as jnp
from jax import lax
from jax.experimental import pallas as pl
from jax.experimental.pallas import tpu as pltpu
from jax.experimental.pallas import tpu_sc as plsc

MAX_SIZE = 100000
STATE_DIM = 128
BATCH = 16384

_NC = 2
_NS = 16
_NW = _NC * _NS
_BPW = BATCH // _NW

_MESH = plsc.VectorSubcoreMesh(core_axis_name="c", subcore_axis_name="s")


@functools.partial(
    pl.kernel,
    mesh=_MESH,
    out_type=(
        jax.ShapeDtypeStruct((BATCH, STATE_DIM), jnp.float32),
        jax.ShapeDtypeStruct((BATCH,), jnp.int32),
        jax.ShapeDtypeStruct((BATCH,), jnp.float32),
        jax.ShapeDtypeStruct((BATCH,), jnp.int32),
    ),
    scratch_types=[
        pltpu.VMEM((_BPW,), jnp.int32),
        pltpu.VMEM((_BPW, STATE_DIM), jnp.float32),
        pltpu.VMEM((_BPW,), jnp.int32),
        pltpu.VMEM((_BPW,), jnp.float32),
        pltpu.VMEM((_BPW,), jnp.int32),
        pltpu.SemaphoreType.DMA,
        pltpu.SemaphoreType.DMA,
    ],
)
def _sample_s(s_hbm, a_hbm, r_hbm, dw_hbm, ind_hbm,
              out_s, out_a, out_r, out_dw,
              idx_v, rows_v, a_v, r_v, dw_v, sem_big, sem_small):
    wid = lax.axis_index("s") * _NC + lax.axis_index("c")
    base = wid * _BPW
    pltpu.sync_copy(ind_hbm.at[pl.ds(base, _BPW)], idx_v)

    ca = pltpu.async_copy(a_hbm.at[idx_v], a_v, sem_small)
    cr = pltpu.async_copy(r_hbm.at[idx_v], r_v, sem_small)
    cd = pltpu.async_copy(dw_hbm.at[idx_v], dw_v, sem_small)

    pltpu.async_copy(s_hbm.at[idx_v], rows_v, sem_big).wait()
    pltpu.sync_copy(rows_v, out_s.at[pl.ds(base, _BPW)])

    ca.wait()
    cr.wait()
    cd.wait()
    pltpu.sync_copy(a_v, out_a.at[pl.ds(base, _BPW)])
    pltpu.sync_copy(r_v, out_r.at[pl.ds(base, _BPW)])
    pltpu.sync_copy(dw_v, out_dw.at[pl.ds(base, _BPW)])


@functools.partial(
    pl.kernel,
    mesh=_MESH,
    out_type=jax.ShapeDtypeStruct((BATCH, STATE_DIM), jnp.float32),
    scratch_types=[
        pltpu.VMEM((_BPW,), jnp.int32),
        pltpu.VMEM((_BPW, STATE_DIM), jnp.float32),
        pltpu.SemaphoreType.DMA,
    ],
)
def _sample_sn(sn_hbm, ind_hbm, out_sn, idx_v, rows_v, sem_big):
    wid = lax.axis_index("s") * _NC + lax.axis_index("c")
    base = wid * _BPW
    pltpu.sync_copy(ind_hbm.at[pl.ds(base, _BPW)], idx_v)
    pltpu.async_copy(sn_hbm.at[idx_v], rows_v, sem_big).wait()
    pltpu.sync_copy(rows_v, out_sn.at[pl.ds(base, _BPW)])


def kernel(s, a, r, s_next, dw, ind):
    sn_b = _sample_sn(s_next, ind)
    s_b, a_b, r_b, dw_b = _sample_s(
        s, a.reshape(MAX_SIZE), r.reshape(MAX_SIZE), dw.reshape(MAX_SIZE),
        ind)
    return (s_b, a_b.reshape(BATCH, 1), r_b.reshape(BATCH, 1), sn_b,
            dw_b.reshape(BATCH, 1))


# speedup vs baseline: 1.1146x; 1.1146x over previous
import functools

import jax
import jax.numpy as jnp
from jax import lax
from jax.experimental import pallas as pl
from jax.experimental.pallas import tpu as pltpu
from jax.experimental.pallas import tpu_sc as plsc

MAX_SIZE = 100000
STATE_DIM = 128
BATCH = 16384

_NC = 2
_NS = 16
_NW = _NC * _NS
_BPW = BATCH // _NW


@functools.partial(
    pl.kernel,
    mesh=plsc.VectorSubcoreMesh(core_axis_name="c", subcore_axis_name="s"),
    out_type=(
        jax.ShapeDtypeStruct((BATCH, STATE_DIM), jnp.float32),
        jax.ShapeDtypeStruct((BATCH,), jnp.int32),
        jax.ShapeDtypeStruct((BATCH,), jnp.float32),
        jax.ShapeDtypeStruct((BATCH, STATE_DIM), jnp.float32),
        jax.ShapeDtypeStruct((BATCH,), jnp.int32),
    ),
    scratch_types=[
        pltpu.VMEM((_BPW,), jnp.int32),
        pltpu.VMEM((_BPW, STATE_DIM), jnp.float32),
        pltpu.VMEM((_BPW,), jnp.int32),
        pltpu.VMEM((_BPW,), jnp.float32),
        pltpu.VMEM((_BPW,), jnp.int32),
        pltpu.SemaphoreType.DMA,
        pltpu.SemaphoreType.DMA,
        pltpu.SemaphoreType.DMA,
    ],
)
def _sample(s_hbm, a_hbm, r_hbm, sn_hbm, dw_hbm, ind_hbm,
            out_s, out_a, out_r, out_sn, out_dw,
            idx_v, rows_v, a_v, r_v, dw_v, sem_g, sem_n, sem_w):
    wid = lax.axis_index("s") * _NC + lax.axis_index("c")
    base = wid * _BPW
    pltpu.sync_copy(ind_hbm.at[pl.ds(base, _BPW)], idx_v)

    ca = pltpu.async_copy(a_hbm.at[idx_v], a_v, sem_n)
    cr = pltpu.async_copy(r_hbm.at[idx_v], r_v, sem_n)
    cd = pltpu.async_copy(dw_hbm.at[idx_v], dw_v, sem_n)
    gs = pltpu.async_copy(s_hbm.at[idx_v], rows_v, sem_g)

    ca.wait()
    cr.wait()
    cd.wait()
    wa = pltpu.async_copy(a_v, out_a.at[pl.ds(base, _BPW)], sem_w)
    wr = pltpu.async_copy(r_v, out_r.at[pl.ds(base, _BPW)], sem_w)
    wd = pltpu.async_copy(dw_v, out_dw.at[pl.ds(base, _BPW)], sem_w)

    gs.wait()
    pltpu.sync_copy(rows_v, out_s.at[pl.ds(base, _BPW)])
    pltpu.async_copy(sn_hbm.at[idx_v], rows_v, sem_g).wait()
    pltpu.sync_copy(rows_v, out_sn.at[pl.ds(base, _BPW)])

    wa.wait()
    wr.wait()
    wd.wait()


def kernel(s, a, r, s_next, dw, ind):
    s_b, a_b, r_b, sn_b, dw_b = _sample(
        s, a.reshape(MAX_SIZE), r.reshape(MAX_SIZE), s_next,
        dw.reshape(MAX_SIZE), ind)
    return (s_b, a_b.reshape(BATCH, 1), r_b.reshape(BATCH, 1), sn_b,
            dw_b.reshape(BATCH, 1))
